# Initial kernel scaffold; baseline (speedup 1.0000x reference)
import functools

import jax
import jax.numpy as jnp
from jax import lax
from jax.experimental import pallas as pl
from jax.experimental.pallas import tpu as pltpu
from jax.experimental.pallas import tpu_sc as plsc

_N = 10000
_E = 320000
_D = 128
_G = 256
_T = 10
_GAMMA = 0.4
_ROWS = 1000
_GRID = _N // _ROWS
_PREC = lax.Precision.HIGHEST

_NC, _NS = 2, 16
_EW = _E // (_NC * _NS)
_K = 80
_CH = _EW // _K
_RS = _N // _NS


def _sc_agg(h, src, dst):
    mesh = plsc.VectorSubcoreMesh(core_axis_name="c", subcore_axis_name="s")

    @functools.partial(
        pl.kernel,
        out_type=jax.ShapeDtypeStruct((_NC, _N, _D), jnp.float32),
        mesh=mesh,
        scratch_types=[
            pltpu.VMEM((_K,), jnp.int32),
            pltpu.VMEM((_K,), jnp.int32),
            pltpu.VMEM((_K, _D), jnp.float32),
            pltpu.VMEM_SHARED((_N, _D), jnp.float32),
            pltpu.SemaphoreType.DMA,
        ],
    )
    def run(h_hbm, src_hbm, dst_hbm, out_hbm, sidx, didx, rows, acc, sem):
        c = lax.axis_index("c")
        s = lax.axis_index("s")
        wid = s * _NC + c
        pltpu.sync_copy(h_hbm.at[pl.ds(s * _RS, _RS)], acc.at[pl.ds(s * _RS, _RS)])
        plsc.subcore_barrier()
        base = wid * _EW

        def body(i, carry):
            off = base + i * _K
            pltpu.sync_copy(src_hbm.at[pl.ds(off, _K)], sidx)
            pltpu.sync_copy(dst_hbm.at[pl.ds(off, _K)], didx)
            pltpu.async_copy(h_hbm.at[sidx], rows, sem).wait()
            pltpu.sync_copy(rows, acc.at[didx], add=True)
            return carry

        lax.fori_loop(0, _CH, body, 0)
        plsc.subcore_barrier()
        pltpu.sync_copy(acc.at[pl.ds(s * _RS, _RS)],
                        out_hbm.at[c, pl.ds(s * _RS, _RS)])

    return run(h, src, dst)


def _stats_update(st_ref, z, i):
    @pl.when(i == 0)
    def _init():
        st_ref[...] = jnp.zeros_like(st_ref)

    s0 = jnp.sum(z, axis=0, keepdims=True)
    s1 = jnp.sum(z * z, axis=0, keepdims=True)
    pad = jnp.zeros((6, z.shape[1]), jnp.float32)
    st_ref[...] += jnp.concatenate([s0, s1, pad], axis=0)


def _bn_coeffs(st, n, g, be):
    m = st[0] / n
    v = st[1] / n - m * m
    inv = g / jnp.sqrt(v + 1e-5)
    return inv.reshape(1, -1), (be - m * inv).reshape(1, -1)


def _first_mm(x, W1, b1):
    twoD = W1.shape[1]

    def body(x_ref, w_ref, b_ref, z_ref, st_ref):
        i = pl.program_id(0)
        z = jnp.dot(x_ref[...], w_ref[...], precision=_PREC,
                    preferred_element_type=jnp.float32) + b_ref[...]
        z_ref[...] = z
        _stats_update(st_ref, z, i)

    return pl.pallas_call(
        body,
        grid=(_GRID,),
        in_specs=[
            pl.BlockSpec((_ROWS, x.shape[1]), lambda i: (i, 0)),
            pl.BlockSpec(W1.shape, lambda i: (0, 0)),
            pl.BlockSpec((1, twoD), lambda i: (0, 0)),
        ],
        out_specs=(
            pl.BlockSpec((_ROWS, twoD), lambda i: (i, 0)),
            pl.BlockSpec((8, twoD), lambda i: (0, 0)),
        ),
        out_shape=(
            jax.ShapeDtypeStruct((_N, twoD), jnp.float32),
            jax.ShapeDtypeStruct((8, twoD), jnp.float32),
        ),
    )(x, W1, b1.reshape(1, -1))


def _gin_mm1(h, parts, em1, W1, b1):
    twoD = W1.shape[1]

    def body(e_ref, h_ref, p_ref, w_ref, b_ref, z_ref, st_ref):
        i = pl.program_id(0)
        zin = e_ref[0, 0] * h_ref[...] + p_ref[0] + p_ref[1]
        z = jnp.dot(zin, w_ref[...], precision=_PREC,
                    preferred_element_type=jnp.float32) + b_ref[...]
        z_ref[...] = z
        _stats_update(st_ref, z, i)

    return pl.pallas_call(
        body,
        grid=(_GRID,),
        in_specs=[
            pl.BlockSpec(memory_space=pltpu.SMEM),
            pl.BlockSpec((_ROWS, _D), lambda i: (i, 0)),
            pl.BlockSpec((2, _ROWS, _D), lambda i: (0, i, 0)),
            pl.BlockSpec((_D, twoD), lambda i: (0, 0)),
            pl.BlockSpec((1, twoD), lambda i: (0, 0)),
        ],
        out_specs=(
            pl.BlockSpec((_ROWS, twoD), lambda i: (i, 0)),
            pl.BlockSpec((8, twoD), lambda i: (0, 0)),
        ),
        out_shape=(
            jax.ShapeDtypeStruct((_N, twoD), jnp.float32),
            jax.ShapeDtypeStruct((8, twoD), jnp.float32),
        ),
    )(em1, h, parts, W1, b1.reshape(1, -1))


def _mid_mm(z1, mult, add, W2, b2):
    twoD = z1.shape[1]
    dout = W2.shape[1]

    def body(m_ref, a_ref, z1_ref, w_ref, b_ref, z2_ref, st_ref):
        i = pl.program_id(0)
        zn = jnp.maximum(z1_ref[...] * m_ref[...] + a_ref[...], 0.0)
        z2 = jnp.dot(zn, w_ref[...], precision=_PREC,
                     preferred_element_type=jnp.float32) + b_ref[...]
        z2_ref[...] = z2
        _stats_update(st_ref, z2, i)

    return pl.pallas_call(
        body,
        grid=(_GRID,),
        in_specs=[
            pl.BlockSpec((1, twoD), lambda i: (0, 0)),
            pl.BlockSpec((1, twoD), lambda i: (0, 0)),
            pl.BlockSpec((_ROWS, twoD), lambda i: (i, 0)),
            pl.BlockSpec((twoD, dout), lambda i: (0, 0)),
            pl.BlockSpec((1, dout), lambda i: (0, 0)),
        ],
        out_specs=(
            pl.BlockSpec((_ROWS, dout), lambda i: (i, 0)),
            pl.BlockSpec((8, dout), lambda i: (0, 0)),
        ),
        out_shape=(
            jax.ShapeDtypeStruct((_N, dout), jnp.float32),
            jax.ShapeDtypeStruct((8, dout), jnp.float32),
        ),
    )(mult, add, z1, W2, b2.reshape(1, -1))


def _bn_res(z2, mult, add, h, relu_out):
    def body(m_ref, a_ref, z_ref, h_ref, o_ref):
        zn = z_ref[...] * m_ref[...] + a_ref[...]
        if relu_out:
            zn = jnp.maximum(zn, 0.0)
        o_ref[...] = zn + h_ref[...]

    return pl.pallas_call(
        body,
        grid=(_GRID,),
        in_specs=[
            pl.BlockSpec((1, _D), lambda i: (0, 0)),
            pl.BlockSpec((1, _D), lambda i: (0, 0)),
            pl.BlockSpec((_ROWS, _D), lambda i: (i, 0)),
            pl.BlockSpec((_ROWS, _D), lambda i: (i, 0)),
        ],
        out_specs=pl.BlockSpec((_ROWS, _D), lambda i: (i, 0)),
        out_shape=jax.ShapeDtypeStruct((_N, _D), jnp.float32),
    )(mult, add, z2, h)


def _gate_mm(z1, mult, add, wdrep, gdrep):
    twoD = z1.shape[1]

    def body(m_ref, a_ref, z1_ref, w_ref, gd_ref, o_ref):
        zn = jnp.maximum(z1_ref[...] * m_ref[...] + a_ref[...], 0.0)
        t = jnp.dot(zn, w_ref[...], precision=_PREC,
                    preferred_element_type=jnp.float32) + gd_ref[...]
        o_ref[...] = 1.0 / (1.0 + jnp.exp(-t))

    return pl.pallas_call(
        body,
        grid=(_GRID,),
        in_specs=[
            pl.BlockSpec((1, twoD), lambda i: (0, 0)),
            pl.BlockSpec((1, twoD), lambda i: (0, 0)),
            pl.BlockSpec((_ROWS, twoD), lambda i: (i, 0)),
            pl.BlockSpec((twoD, _D), lambda i: (0, 0)),
            pl.BlockSpec((_ROWS, _D), lambda i: (i, 0)),
        ],
        out_specs=pl.BlockSpec((_ROWS, _D), lambda i: (i, 0)),
        out_shape=jax.ShapeDtypeStruct((_N, _D), jnp.float32),
    )(mult, add, z1, wdrep, gdrep)


def _pool(h, gate, batch3):
    def body(h_ref, g_ref, b_ref, o_ref):
        i = pl.program_id(0)

        @pl.when(i == 0)
        def _init():
            o_ref[...] = jnp.zeros_like(o_ref)

        h = h_ref[...]
        g = g_ref[...]
        b = b_ref[0]
        onehot = (lax.broadcasted_iota(jnp.int32, (_G, _ROWS), 0) == b
                  ).astype(jnp.float32)
        vals = jnp.concatenate([h, g * h, g, jnp.ones_like(g)], axis=1)
        o_ref[...] += jnp.dot(onehot, vals, precision=_PREC,
                              preferred_element_type=jnp.float32)

    return pl.pallas_call(
        body,
        grid=(_GRID,),
        in_specs=[
            pl.BlockSpec((_ROWS, _D), lambda i: (i, 0)),
            pl.BlockSpec((_ROWS, _D), lambda i: (i, 0)),
            pl.BlockSpec((1, 1, _ROWS), lambda i: (i, 0, 0)),
        ],
        out_specs=pl.BlockSpec((_G, 4 * _D), lambda i: (0, 0)),
        out_shape=jax.ShapeDtypeStruct((_G, 4 * _D), jnp.float32),
    )(h, gate, batch3)


def _final(pool_out, pW1, pb1, pg, pbe, pW2p, pb2p):
    twoD = pW1.shape[1]

    def body(po_ref, w1_ref, b1_ref, g_ref, be_ref, w2_ref, b2_ref,
             pred_ref, loss_ref):
        po = po_ref[...]
        sg = po[:, 2 * _D:2 * _D + 1]
        cnt = po[:, 3 * _D:3 * _D + 1]
        c = jnp.maximum(cnt, 1.0)
        ho = po[:, :_D] / c
        hr = po[:, _D:2 * _D] / c
        rn = sg + 1e-8
        en = (cnt - sg) + 1e-8
        loss = jnp.mean(jnp.abs(rn / (rn + en) - _GAMMA))
        loss_ref[...] = jnp.full((8, _D), loss, jnp.float32)

        def mlp(v):
            z1 = jnp.dot(v, w1_ref[...], precision=_PREC,
                         preferred_element_type=jnp.float32) + b1_ref[...]
            m = jnp.mean(z1, axis=0, keepdims=True)
            var = jnp.mean((z1 - m) ** 2, axis=0, keepdims=True)
            zn = (z1 - m) / jnp.sqrt(var + 1e-5) * g_ref[...] + be_ref[...]
            zn = jnp.maximum(zn, 0.0)
            return jnp.dot(zn, w2_ref[...], precision=_PREC,
                           preferred_element_type=jnp.float32) + b2_ref[...]

        pred_ref[0] = mlp(ho)
        pred_ref[1] = mlp(hr)

    return pl.pallas_call(
        body,
        in_specs=[
            pl.BlockSpec((_G, 4 * _D), lambda: (0, 0)),
            pl.BlockSpec((_D, twoD), lambda: (0, 0)),
            pl.BlockSpec((1, twoD), lambda: (0, 0)),
            pl.BlockSpec((1, twoD), lambda: (0, 0)),
            pl.BlockSpec((1, twoD), lambda: (0, 0)),
            pl.BlockSpec((twoD, _D), lambda: (0, 0)),
            pl.BlockSpec((1, _D), lambda: (0, 0)),
        ],
        out_specs=(
            pl.BlockSpec((2, _G, _D), lambda: (0, 0, 0)),
            pl.BlockSpec((8, _D), lambda: (0, 0)),
        ),
        out_shape=(
            jax.ShapeDtypeStruct((2, _G, _D), jnp.float32),
            jax.ShapeDtypeStruct((8, _D), jnp.float32),
        ),
    )(pool_out, pW1, pb1, pg, pbe, pW2p, pb2p)


def _gin_layer(h, parts, p, pre, l, relu_out):
    em1 = (p[pre + 'eps'][l] - 1.0).reshape(1, 1)
    z1, st1 = _gin_mm1(h, parts, em1, p[pre + 'W1'][l], p[pre + 'b1'][l])
    mult1, add1 = _bn_coeffs(st1, _N, p[pre + 'g1'][l], p[pre + 'be1'][l])
    z2, st2 = _mid_mm(z1, mult1, add1, p[pre + 'W2'][l], p[pre + 'b2'][l])
    mult2, add2 = _bn_coeffs(st2, _N, p[pre + 'g'][l], p[pre + 'be'][l])
    return _bn_res(z2, mult2, add2, h, relu_out)


def kernel(x, edge_index, batch, params):
    src = edge_index[0]
    dst = edge_index[1]
    batch3 = batch.reshape(_GRID, 1, _ROWS)
    aggx = _sc_agg(x, src, dst)

    h = x
    parts = aggx
    for l in range(5):
        h = _gin_layer(h, parts, params, 'e', l, relu_out=(l < 4))
        if l < 4:
            parts = _sc_agg(h, src, dst)
    h_node = h

    xr = x
    parts = aggx
    for l in range(2):
        xr = _gin_layer(xr, parts, params, 'r', l, relu_out=(l < 1))
        if l < 1:
            parts = _sc_agg(xr, src, dst)

    z1g, stg = _first_mm(xr, params['gW1'], params['gb1'])
    multg, addg = _bn_coeffs(stg, _N, params['gg'], params['gbe'])
    wd = params['gW2'][:, 1] - params['gW2'][:, 0]
    bd = params['gb2'][1] - params['gb2'][0]
    u = jax.random.uniform(jax.random.key(42), (_N, 2),
                           minval=1e-6, maxval=1.0 - 1e-6)
    gum = -jnp.log(-jnp.log(u))
    gd = gum[:, 1] - gum[:, 0] + bd
    wdrep = jnp.broadcast_to(wd[:, None], (2 * _D, _D))
    gdrep = jnp.broadcast_to(gd[:, None], (_N, _D))
    gate = _gate_mm(z1g, multg, addg, wdrep, gdrep)

    pool_out = _pool(h_node, gate, batch3)

    pW2p = jnp.pad(params['pW2'], ((0, 0), (0, _D - _T)))
    pb2p = jnp.pad(params['pb2'], (0, _D - _T)).reshape(1, -1)
    preds, lossout = _final(pool_out, params['pW1'],
                            params['pb1'].reshape(1, -1),
                            params['pg'].reshape(1, -1),
                            params['pbe'].reshape(1, -1), pW2p, pb2p)

    return (preds[0, :, :_T], preds[1, :, :_T], lossout[0, 0])


# speedup vs baseline: 2.8146x; 2.8146x over previous
import functools

import jax
import jax.numpy as jnp
from jax import lax
from jax.experimental import pallas as pl
from jax.experimental.pallas import tpu as pltpu
from jax.experimental.pallas import tpu_sc as plsc

_N = 10000
_E = 320000
_D = 128
_G = 256
_T = 10
_GAMMA = 0.4
_ROWS = 1000
_GRID = _N // _ROWS
_PREC = lax.Precision.DEFAULT

_NC, _NS = 2, 16
_K = 80
_AROWS = _N + _NS
_ZR = 48
_WB = 624
_WTAIL = _AROWS - _WB * _NS

_BH = [10080 * k for k in range(12)] + [120720, 130560, 140400, 150240, 160000]
_BOUNDS = _BH + [160000 + b for b in _BH[1:]]


def _sc_agg(h, es, ed2, mi):
    mesh = plsc.VectorSubcoreMesh(core_axis_name="c", subcore_axis_name="s")

    @functools.partial(
        pl.kernel,
        out_type=jax.ShapeDtypeStruct((_NC, _AROWS, _D), jnp.float32),
        mesh=mesh,
        scratch_types=[
            pltpu.VMEM((_K,), jnp.int32),
            pltpu.VMEM((_K,), jnp.int32),
            pltpu.VMEM((_K, _D), jnp.float32),
            pltpu.VMEM((_ZR, _D), jnp.float32),
            pltpu.VMEM((_NS,), jnp.int32),
            pltpu.VMEM((_NS, _D), jnp.float32),
            pltpu.VMEM_SHARED((_AROWS, _D), jnp.float32),
            pltpu.SemaphoreType.DMA,
        ],
    )
    def run(h_hbm, es_hbm, ed_hbm, mi_hbm, out_hbm,
            sidx, didx, rows, zbuf, midx, mrows, acc, sem):
        c = lax.axis_index("c")
        s = lax.axis_index("s")
        zv = jnp.zeros((16,), jnp.float32)
        for r in range(_ZR):
            for j in range(_D // 16):
                zbuf[r, pl.ds(16 * j, 16)] = zv
        for j in range(_WB // _ZR):
            pltpu.sync_copy(zbuf, acc.at[pl.ds(s * _WB + j * _ZR, _ZR)])

        @pl.when(s == _NS - 1)
        def _ztail():
            pltpu.sync_copy(zbuf.at[pl.ds(0, _WTAIL)],
                            acc.at[pl.ds(_WB * _NS, _WTAIL)])

        plsc.subcore_barrier()

        base = c * (_E // _NC) + jnp.where(
            s <= 11, 10080 * s, 110880 + 9840 * (s - 11))
        nch = jnp.where(s < 11, 126, jnp.where(s < 15, 123, 122))

        def body(i, carry):
            off = base + i * _K
            pltpu.sync_copy(es_hbm.at[pl.ds(off, _K)], sidx)
            pltpu.sync_copy(ed_hbm.at[pl.ds(off, _K)], didx)
            pltpu.async_copy(h_hbm.at[sidx], rows, sem).wait()
            pltpu.sync_copy(rows, acc.at[didx], add=True)
            return carry

        lax.fori_loop(0, nch, body, 0)
        plsc.subcore_barrier()

        @pl.when(s == 0)
        def _merge():
            pltpu.sync_copy(mi_hbm.at[c], midx)
            pltpu.sync_copy(acc.at[pl.ds(_N, _NS)], mrows)
            pltpu.sync_copy(mrows, acc.at[midx], add=True)

        plsc.subcore_barrier()
        pltpu.sync_copy(acc.at[pl.ds(s * _WB, _WB)],
                        out_hbm.at[c, pl.ds(s * _WB, _WB)])

        @pl.when(s == _NS - 1)
        def _wtail():
            pltpu.sync_copy(acc.at[pl.ds(_WB * _NS, _WTAIL)],
                            out_hbm.at[c, pl.ds(_WB * _NS, _WTAIL)])

    return run(h, es, ed2, mi)


def _mm(xv, W, b):
    Kd = xv.shape[1]
    Do = W.shape[1]

    def body(x_ref, w_ref, b_ref, o_ref):
        o_ref[...] = jnp.dot(x_ref[...], w_ref[...], precision=_PREC,
                             preferred_element_type=jnp.float32) + b_ref[...]

    return pl.pallas_call(
        body,
        grid=(_GRID,),
        in_specs=[
            pl.BlockSpec((_ROWS, Kd), lambda i: (i, 0)),
            pl.BlockSpec((Kd, Do), lambda i: (0, 0)),
            pl.BlockSpec((1, Do), lambda i: (0, 0)),
        ],
        out_specs=pl.BlockSpec((_ROWS, Do), lambda i: (i, 0)),
        out_shape=jax.ShapeDtypeStruct((_N, Do), jnp.float32),
    )(xv, W, b.reshape(1, -1))


def _gate_mm(zn, wdrep, gdrep):
    twoD = zn.shape[1]

    def body(z_ref, w_ref, gd_ref, o_ref):
        t = jnp.dot(z_ref[...], w_ref[...], precision=_PREC,
                    preferred_element_type=jnp.float32) + gd_ref[...]
        o_ref[...] = 1.0 / (1.0 + jnp.exp(-t))

    return pl.pallas_call(
        body,
        grid=(_GRID,),
        in_specs=[
            pl.BlockSpec((_ROWS, twoD), lambda i: (i, 0)),
            pl.BlockSpec((twoD, _D), lambda i: (0, 0)),
            pl.BlockSpec((_ROWS, _D), lambda i: (i, 0)),
        ],
        out_specs=pl.BlockSpec((_ROWS, _D), lambda i: (i, 0)),
        out_shape=jax.ShapeDtypeStruct((_N, _D), jnp.float32),
    )(zn, wdrep, gdrep)


def _pool(h, gate, batch3):
    def body(h_ref, g_ref, b_ref, o_ref):
        i = pl.program_id(0)

        @pl.when(i == 0)
        def _init():
            o_ref[...] = jnp.zeros_like(o_ref)

        h = h_ref[...]
        g = g_ref[...]
        b = b_ref[0]
        onehot = (lax.broadcasted_iota(jnp.int32, (_G, _ROWS), 0) == b
                  ).astype(jnp.float32)
        vals = jnp.concatenate([h, g * h, g, jnp.ones_like(g)], axis=1)
        o_ref[...] += jnp.dot(onehot, vals, precision=lax.Precision.HIGHEST,
                              preferred_element_type=jnp.float32)

    return pl.pallas_call(
        body,
        grid=(_GRID,),
        in_specs=[
            pl.BlockSpec((_ROWS, _D), lambda i: (i, 0)),
            pl.BlockSpec((_ROWS, _D), lambda i: (i, 0)),
            pl.BlockSpec((1, 1, _ROWS), lambda i: (i, 0, 0)),
        ],
        out_specs=pl.BlockSpec((_G, 4 * _D), lambda i: (0, 0)),
        out_shape=jax.ShapeDtypeStruct((_G, 4 * _D), jnp.float32),
    )(h, gate, batch3)


def _final(pool_out, pW1, pb1, pg, pbe, pW2p, pb2p):
    twoD = pW1.shape[1]

    def body(po_ref, w1_ref, b1_ref, g_ref, be_ref, w2_ref, b2_ref,
             pred_ref, loss_ref):
        po = po_ref[...]
        sg = po[:, 2 * _D:2 * _D + 1]
        cnt = po[:, 3 * _D:3 * _D + 1]
        c = jnp.maximum(cnt, 1.0)
        ho = po[:, :_D] / c
        hr = po[:, _D:2 * _D] / c
        rn = sg + 1e-8
        en = (cnt - sg) + 1e-8
        loss = jnp.mean(jnp.abs(rn / (rn + en) - _GAMMA))
        loss_ref[...] = jnp.full((8, _D), loss, jnp.float32)

        def mlp(v):
            z1 = jnp.dot(v, w1_ref[...], precision=_PREC,
                         preferred_element_type=jnp.float32) + b1_ref[...]
            m = jnp.mean(z1, axis=0, keepdims=True)
            var = jnp.mean((z1 - m) ** 2, axis=0, keepdims=True)
            zn = (z1 - m) / jnp.sqrt(var + 1e-5) * g_ref[...] + be_ref[...]
            zn = jnp.maximum(zn, 0.0)
            return jnp.dot(zn, w2_ref[...], precision=_PREC,
                           preferred_element_type=jnp.float32) + b2_ref[...]

        pred_ref[0] = mlp(ho)
        pred_ref[1] = mlp(hr)

    return pl.pallas_call(
        body,
        in_specs=[
            pl.BlockSpec((_G, 4 * _D), lambda: (0, 0)),
            pl.BlockSpec((_D, twoD), lambda: (0, 0)),
            pl.BlockSpec((1, twoD), lambda: (0, 0)),
            pl.BlockSpec((1, twoD), lambda: (0, 0)),
            pl.BlockSpec((1, twoD), lambda: (0, 0)),
            pl.BlockSpec((twoD, _D), lambda: (0, 0)),
            pl.BlockSpec((1, _D), lambda: (0, 0)),
        ],
        out_specs=(
            pl.BlockSpec((2, _G, _D), lambda: (0, 0, 0)),
            pl.BlockSpec((8, _D), lambda: (0, 0)),
        ),
        out_shape=(
            jax.ShapeDtypeStruct((2, _G, _D), jnp.float32),
            jax.ShapeDtypeStruct((8, _D), jnp.float32),
        ),
    )(pool_out, pW1, pb1, pg, pbe, pW2p, pb2p)


def _bn_x(z, g, be):
    m = z.mean(axis=0)
    v = z.var(axis=0)
    return (z - m) / jnp.sqrt(v + 1e-5) * g + be


def _gin_layer(h, agg, p, pre, l, relu_out):
    zin = (1.0 + p[pre + 'eps'][l]) * h + agg
    z1 = _mm(zin, p[pre + 'W1'][l], p[pre + 'b1'][l])
    zn = jax.nn.relu(_bn_x(z1, p[pre + 'g1'][l], p[pre + 'be1'][l]))
    z2 = _mm(zn, p[pre + 'W2'][l], p[pre + 'b2'][l])
    z = _bn_x(z2, p[pre + 'g'][l], p[pre + 'be'][l])
    if relu_out:
        z = jax.nn.relu(z)
    return z + h


def kernel(x, edge_index, batch, params):
    src = edge_index[0]
    dst = edge_index[1]
    batch3 = batch.reshape(_GRID, 1, _ROWS)

    perm = jnp.argsort(dst, stable=True)
    es = src[perm].astype(jnp.int32)
    ed = dst[perm].astype(jnp.int32)
    bl = jnp.asarray(_BOUNDS, jnp.int32)
    pos = jnp.arange(_E, dtype=jnp.int32)
    k_of = jnp.searchsorted(bl, pos, side='right').astype(jnp.int32) - 1
    hn = ed[bl[:-1]]
    he = jnp.minimum(bl[1:],
                     jnp.searchsorted(ed, hn, side='right').astype(jnp.int32))
    is_head = pos < he[k_of]
    ed2 = jnp.where(is_head, _N + (k_of % _NS), ed).astype(jnp.int32)
    mi = hn.reshape(_NC, _NS)

    def agg_of(h):
        o = _sc_agg(h, es, ed2, mi)
        return o[0, :_N] + o[1, :_N]

    aggx = agg_of(x)

    h = x
    agg = aggx
    for l in range(5):
        h = _gin_layer(h, agg, params, 'e', l, relu_out=(l < 4))
        if l < 4:
            agg = agg_of(h)
    h_node = h

    xr = x
    agg = aggx
    for l in range(2):
        xr = _gin_layer(xr, agg, params, 'r', l, relu_out=(l < 1))
        if l < 1:
            agg = agg_of(xr)

    z1g = _mm(xr, params['gW1'], params['gb1'])
    zng = jax.nn.relu(_bn_x(z1g, params['gg'], params['gbe']))
    wd = params['gW2'][:, 1] - params['gW2'][:, 0]
    bd = params['gb2'][1] - params['gb2'][0]
    u = jax.random.uniform(jax.random.key(42), (_N, 2),
                           minval=1e-6, maxval=1.0 - 1e-6)
    gum = -jnp.log(-jnp.log(u))
    gd = gum[:, 1] - gum[:, 0] + bd
    wdrep = jnp.broadcast_to(wd[:, None], (2 * _D, _D))
    gdrep = jnp.broadcast_to(gd[:, None], (_N, _D))
    gate = _gate_mm(zng, wdrep, gdrep)

    pool_out = _pool(h_node, gate, batch3)

    pW2p = jnp.pad(params['pW2'], ((0, 0), (0, _D - _T)))
    pb2p = jnp.pad(params['pb2'], (0, _D - _T)).reshape(1, -1)
    preds, lossout = _final(pool_out, params['pW1'],
                            params['pb1'].reshape(1, -1),
                            params['pg'].reshape(1, -1),
                            params['pbe'].reshape(1, -1), pW2p, pb2p)

    return (preds[0, :, :_T], preds[1, :, :_T], lossout[0, 0])
